# Initial kernel scaffold; baseline (speedup 1.0000x reference)
import jax
import jax.numpy as jnp
from jax.experimental import pallas as pl


def kernel(im, defgrad, affine):
    raise NotImplementedError("write your pallas kernel here")


# speedup vs baseline: 8.2320x; 8.2320x over previous
import functools

import jax
import jax.numpy as jnp
from jax import lax
from jax.experimental import pallas as pl
from jax.experimental.pallas import tpu as pltpu
from jax.experimental.pallas import tpu_sc as plsc

B, X, Y, Z = 2, 64, 64, 64
N = B * X * Y * Z
NR = B * X * Y
RB = 512
NWORK = 32
PERW = N // NWORK
CH = 2048
NCH = PERW // CH


def _table_body(a00_ref, a01_ref, a10_ref, a11_ref, p_ref, out_ref):
    def zshift(m):
        return jnp.concatenate([m[:, 1:], m[:, Z - 1:Z]], axis=1)

    a00 = a00_ref[...]
    a01 = a01_ref[...]
    a10 = a10_ref[...]
    a11 = a11_ref[...]
    corners = [a00, zshift(a00), a01, zshift(a01),
               a10, zshift(a10), a11, zshift(a11)]
    acc = jnp.zeros((a00.shape[0], 8 * Z), jnp.float32)
    for c in range(8):
        acc += jnp.dot(corners[c], p_ref[c * Z:(c + 1) * Z, :],
                       preferred_element_type=jnp.float32,
                       precision=lax.Precision.HIGHEST)
    out_ref[...] = acc


def _build_table(im):
    im2 = im.reshape(NR, Z)
    pad = jnp.broadcast_to(im2[NR - 1:NR], (Y + 1, Z))
    im_pad = jnp.concatenate([im2, pad], axis=0)
    a00 = im_pad[0:NR]
    a01 = im_pad[1:NR + 1]
    a10 = im_pad[Y:NR + Y]
    a11 = im_pad[Y + 1:NR + Y + 1]
    rows = jnp.arange(8 * Z, dtype=jnp.int32)
    cols = 8 * (rows % Z) + rows // Z
    perm = jax.nn.one_hot(cols, 8 * Z, dtype=jnp.float32)
    return pl.pallas_call(
        _table_body,
        grid=(NR // RB,),
        in_specs=[pl.BlockSpec((RB, Z), lambda i: (i, 0))] * 4
        + [pl.BlockSpec((8 * Z, 8 * Z), lambda i: (0, 0))],
        out_specs=pl.BlockSpec((RB, 8 * Z), lambda i: (i, 0)),
        out_shape=jax.ShapeDtypeStruct((NR, 8 * Z), jnp.float32),
    )(a00, a01, a10, a11, perm)


def _grid_body(aff_ref, d0_ref, d1_ref, d2_ref,
               gx_ref, gy_ref, gz_ref, xw_ref, yw_ref, zw_ref, idx_ref,
               xcarry):
    b = pl.program_id(0)
    x = pl.program_id(1)

    s0 = 2.0 / (1.0 + jnp.exp(-d0_ref[0, 0]))
    s1 = 2.0 / (1.0 + jnp.exp(-d1_ref[0, 0]))
    s2 = 2.0 / (1.0 + jnp.exp(-d2_ref[0, 0]))

    @pl.when(x == 0)
    def _():
        xcarry[...] = jnp.zeros((Y, Z), jnp.float32)

    xs = xcarry[...] + s0
    xcarry[...] = xs

    ii = lax.broadcasted_iota(jnp.int32, (Y, Z), 0)
    jj = lax.broadcasted_iota(jnp.int32, (Y, Z), 1)
    ltri = (ii >= jj).astype(jnp.float32)
    utri = (ii <= jj).astype(jnp.float32)
    ys = jnp.dot(ltri, s1, preferred_element_type=jnp.float32,
                 precision=lax.Precision.HIGHEST)
    zs = jnp.dot(s2, utri, preferred_element_type=jnp.float32,
                 precision=lax.Precision.HIGHEST)

    a = [aff_ref[0, 0, j] for j in range(12)]

    def rb(v):
        return v.astype(jnp.bfloat16).astype(jnp.float32)

    one = jnp.ones((Y, Z), jnp.float32)

    def rbs(coef):
        return rb(coef * one)

    xsb, ysb, zsb = rb(xs), rb(ys), rb(zs)
    gx = rbs(1.0 + a[0]) * xsb + rbs(a[1]) * ysb + rbs(a[2]) * zsb + rbs(a[3])
    gy = rbs(a[4]) * xsb + rbs(1.0 + a[5]) * ysb + rbs(a[6]) * zsb + rbs(a[7])
    gz = rbs(a[8]) * xsb + rbs(a[9]) * ysb + rbs(1.0 + a[10]) * zsb + rbs(a[11])

    def floor_weight(g):
        f = jnp.floor(g)
        i = f.astype(jnp.int32)
        in_bounds = (i >= 0) & (i <= X - 2)
        ic = jnp.clip(i, 0, X - 1)
        w = jnp.where(in_bounds, g - f, 0.0)
        return ic, w

    x0, xw = floor_weight(gx)
    y0, yw = floor_weight(gy)
    z0, zw = floor_weight(gz)

    gx_ref[0, 0] = gx
    gy_ref[0, 0] = gy
    gz_ref[0, 0] = gz
    xw_ref[0, 0] = xw
    yw_ref[0, 0] = yw
    zw_ref[0, 0] = zw
    idx_ref[0, 0] = b * (X * Y * Z) + x0 * (Y * Z) + y0 * Z + z0


def _grid_fields(affine, d0, d1, d2):
    vol = jax.ShapeDtypeStruct((B, X, Y, Z), jnp.float32)
    ivol = jax.ShapeDtypeStruct((B, X, Y, Z), jnp.int32)
    blk = lambda b, x: (b, x, 0, 0)
    return pl.pallas_call(
        _grid_body,
        grid=(B, X),
        in_specs=[
            pl.BlockSpec((1, 1, 12), lambda b, x: (b, 0, 0),
                         memory_space=pltpu.SMEM),
            pl.BlockSpec((1, 1, Y, Z), blk),
            pl.BlockSpec((1, 1, Y, Z), blk),
            pl.BlockSpec((1, 1, Y, Z), blk),
        ],
        out_specs=[pl.BlockSpec((1, 1, Y, Z), blk)] * 7,
        out_shape=[vol, vol, vol, vol, vol, vol, ivol],
        scratch_shapes=[pltpu.VMEM((Y, Z), jnp.float32)],
    )(affine.reshape(B, 1, 12), d0, d1, d2)


def _sc_body(tab_hbm, idx_hbm, xw_hbm, yw_hbm, zw_hbm, out_hbm,
             idx_v, rows_v, xw_v, yw_v, zw_v, out_v, sem):
    c = lax.axis_index("c")
    s = lax.axis_index("s")
    wid = s * 2 + c

    def chunk(ci, carry):
        base = wid * PERW + ci * CH
        pltpu.sync_copy(idx_hbm.at[pl.ds(base, CH)], idx_v)
        gat = pltpu.async_copy(tab_hbm.at[idx_v], rows_v, sem)
        pltpu.sync_copy(xw_hbm.at[pl.ds(base, CH)], xw_v)
        pltpu.sync_copy(yw_hbm.at[pl.ds(base, CH)], yw_v)
        pltpu.sync_copy(zw_hbm.at[pl.ds(base, CH)], zw_v)
        gat.wait()

        def grp(g, cc):
            o = g * 16
            xw = xw_v[pl.ds(o, 16)]
            yw = yw_v[pl.ds(o, 16)]
            zw = zw_v[pl.ds(o, 16)]
            pid = lax.iota(jnp.int32, 16) + o
            v = [plsc.load_gather(rows_v, [pid, jnp.full((16,), k, jnp.int32)])
                 for k in range(8)]
            m00 = v[0] + zw * (v[1] - v[0])
            m01 = v[2] + zw * (v[3] - v[2])
            m10 = v[4] + zw * (v[5] - v[4])
            m11 = v[6] + zw * (v[7] - v[6])
            n0 = m00 + yw * (m01 - m00)
            n1 = m10 + yw * (m11 - m10)
            out_v[pl.ds(o, 16)] = n0 + xw * (n1 - n0)
            return cc

        lax.fori_loop(0, CH // 16, grp, 0)
        pltpu.sync_copy(out_v, out_hbm.at[pl.ds(base, CH)])
        return carry

    lax.fori_loop(0, NCH, chunk, 0)


@functools.cache
def _sc_blend_kernel():
    return pl.kernel(
        _sc_body,
        out_type=jax.ShapeDtypeStruct((N,), jnp.float32),
        mesh=plsc.VectorSubcoreMesh(core_axis_name="c", subcore_axis_name="s"),
        scratch_types=[
            pltpu.VMEM((CH,), jnp.int32),
            pltpu.VMEM((CH, 8), jnp.float32),
            pltpu.VMEM((CH,), jnp.float32),
            pltpu.VMEM((CH,), jnp.float32),
            pltpu.VMEM((CH,), jnp.float32),
            pltpu.VMEM((CH,), jnp.float32),
            pltpu.SemaphoreType.DMA,
        ],
        compiler_params=pltpu.CompilerParams(needs_layout_passes=False,
                                             use_tc_tiling_on_sc=False),
    )


def kernel(im, defgrad, affine):
    table = _build_table(im)
    d0 = defgrad[..., 0]
    d1 = defgrad[..., 1]
    d2 = defgrad[..., 2]
    gx, gy, gz, xw, yw, zw, idx = _grid_fields(affine, d0, d1, d2)
    out = _sc_blend_kernel()(table.reshape(N, 8), idx.reshape(N),
                             xw.reshape(N), yw.reshape(N), zw.reshape(N))
    grid = jnp.stack([gx, gy, gz], axis=-1)
    return out.reshape(B, X, Y, Z, 1), grid


# speedup vs baseline: 8.2422x; 1.0012x over previous
import functools

import jax
import jax.numpy as jnp
from jax import lax
from jax.experimental import pallas as pl
from jax.experimental.pallas import tpu as pltpu
from jax.experimental.pallas import tpu_sc as plsc

B, X, Y, Z = 2, 64, 64, 64
N = B * X * Y * Z
NR = B * X * Y
RB = 512
NWORK = 32
PERW = N // NWORK
CH = 2048
NCH = PERW // CH


def _table_body(a00_ref, a01_ref, a10_ref, a11_ref, p_ref, out_ref):
    def zshift(m):
        return jnp.concatenate([m[:, 1:], m[:, Z - 1:Z]], axis=1)

    a00 = a00_ref[...]
    a01 = a01_ref[...]
    a10 = a10_ref[...]
    a11 = a11_ref[...]
    corners = [a00, zshift(a00), a01, zshift(a01),
               a10, zshift(a10), a11, zshift(a11)]
    acc = jnp.zeros((a00.shape[0], 8 * Z), jnp.float32)
    for c in range(8):
        acc += jnp.dot(corners[c], p_ref[c * Z:(c + 1) * Z, :],
                       preferred_element_type=jnp.float32,
                       precision=lax.Precision.HIGHEST)
    out_ref[...] = acc


def _build_table(im):
    im2 = im.reshape(NR, Z)
    pad = jnp.broadcast_to(im2[NR - 1:NR], (Y + 1, Z))
    im_pad = jnp.concatenate([im2, pad], axis=0)
    a00 = im_pad[0:NR]
    a01 = im_pad[1:NR + 1]
    a10 = im_pad[Y:NR + Y]
    a11 = im_pad[Y + 1:NR + Y + 1]
    rows = jnp.arange(8 * Z, dtype=jnp.int32)
    cols = 8 * (rows % Z) + rows // Z
    perm = jax.nn.one_hot(cols, 8 * Z, dtype=jnp.float32)
    return pl.pallas_call(
        _table_body,
        grid=(NR // RB,),
        in_specs=[pl.BlockSpec((RB, Z), lambda i: (i, 0))] * 4
        + [pl.BlockSpec((8 * Z, 8 * Z), lambda i: (0, 0))],
        out_specs=pl.BlockSpec((RB, 8 * Z), lambda i: (i, 0)),
        out_shape=jax.ShapeDtypeStruct((NR, 8 * Z), jnp.float32),
    )(a00, a01, a10, a11, perm)


def _grid_body(aff_ref, d0_ref, d1_ref, d2_ref,
               gx_ref, gy_ref, gz_ref, xw_ref, yw_ref, zw_ref, idx_ref,
               xcarry):
    b = pl.program_id(0)
    x = pl.program_id(1)

    s0 = 2.0 / (1.0 + jnp.exp(-d0_ref[0, 0]))
    s1 = 2.0 / (1.0 + jnp.exp(-d1_ref[0, 0]))
    s2 = 2.0 / (1.0 + jnp.exp(-d2_ref[0, 0]))

    @pl.when(x == 0)
    def _():
        xcarry[...] = jnp.zeros((Y, Z), jnp.float32)

    xs = xcarry[...] + s0
    xcarry[...] = xs

    ii = lax.broadcasted_iota(jnp.int32, (Y, Z), 0)
    jj = lax.broadcasted_iota(jnp.int32, (Y, Z), 1)
    ltri = (ii >= jj).astype(jnp.float32)
    utri = (ii <= jj).astype(jnp.float32)
    ys = jnp.dot(ltri, s1, preferred_element_type=jnp.float32,
                 precision=lax.Precision.HIGHEST)
    zs = jnp.dot(s2, utri, preferred_element_type=jnp.float32,
                 precision=lax.Precision.HIGHEST)

    a = [aff_ref[0, 0, j] for j in range(12)]

    def rb(v):
        return v.astype(jnp.bfloat16).astype(jnp.float32)

    one = jnp.ones((Y, Z), jnp.float32)

    def rbs(coef):
        return rb(coef * one)

    xsb, ysb, zsb = rb(xs), rb(ys), rb(zs)
    gx = rbs(1.0 + a[0]) * xsb + rbs(a[1]) * ysb + rbs(a[2]) * zsb + rbs(a[3])
    gy = rbs(a[4]) * xsb + rbs(1.0 + a[5]) * ysb + rbs(a[6]) * zsb + rbs(a[7])
    gz = rbs(a[8]) * xsb + rbs(a[9]) * ysb + rbs(1.0 + a[10]) * zsb + rbs(a[11])

    def floor_weight(g):
        f = jnp.floor(g)
        i = f.astype(jnp.int32)
        in_bounds = (i >= 0) & (i <= X - 2)
        ic = jnp.clip(i, 0, X - 1)
        w = jnp.where(in_bounds, g - f, 0.0)
        return ic, w

    x0, xw = floor_weight(gx)
    y0, yw = floor_weight(gy)
    z0, zw = floor_weight(gz)

    gx_ref[0, 0] = gx
    gy_ref[0, 0] = gy
    gz_ref[0, 0] = gz
    xw_ref[0, 0] = xw
    yw_ref[0, 0] = yw
    zw_ref[0, 0] = zw
    idx_ref[0, 0] = b * (X * Y * Z) + x0 * (Y * Z) + y0 * Z + z0


def _grid_fields(affine, d0, d1, d2):
    vol = jax.ShapeDtypeStruct((B, X, Y, Z), jnp.float32)
    ivol = jax.ShapeDtypeStruct((B, X, Y, Z), jnp.int32)
    blk = lambda b, x: (b, x, 0, 0)
    return pl.pallas_call(
        _grid_body,
        grid=(B, X),
        in_specs=[
            pl.BlockSpec((1, 1, 12), lambda b, x: (b, 0, 0),
                         memory_space=pltpu.SMEM),
            pl.BlockSpec((1, 1, Y, Z), blk),
            pl.BlockSpec((1, 1, Y, Z), blk),
            pl.BlockSpec((1, 1, Y, Z), blk),
        ],
        out_specs=[pl.BlockSpec((1, 1, Y, Z), blk)] * 7,
        out_shape=[vol, vol, vol, vol, vol, vol, ivol],
        scratch_shapes=[pltpu.VMEM((Y, Z), jnp.float32)],
    )(affine.reshape(B, 1, 12), d0, d1, d2)


def _sc_body(tab_hbm, idx_hbm, xw_hbm, yw_hbm, zw_hbm, out_hbm, *scr):
    idx_v = scr[0:2]
    rows_v = scr[2:4]
    xw_v = scr[4:6]
    yw_v = scr[6:8]
    zw_v = scr[8:10]
    out_v = scr[10:12]
    semG = scr[12:14]
    semW = scr[14:16]
    semO = scr[16:18]

    c = lax.axis_index("c")
    s = lax.axis_index("s")
    wid = s * 2 + c
    base0 = wid * PERW

    gdescs = [None, None]
    wdescs = [None, None]
    odescs = [None, None]

    def start_inputs(i):
        p = i % 2
        b = base0 + i * CH
        pltpu.sync_copy(idx_hbm.at[pl.ds(b, CH)], idx_v[p])
        gdescs[p] = pltpu.async_copy(tab_hbm.at[idx_v[p]], rows_v[p], semG[p])
        wdescs[p] = [
            pltpu.async_copy(xw_hbm.at[pl.ds(b, CH)], xw_v[p], semW[p]),
            pltpu.async_copy(yw_hbm.at[pl.ds(b, CH)], yw_v[p], semW[p]),
            pltpu.async_copy(zw_hbm.at[pl.ds(b, CH)], zw_v[p], semW[p]),
        ]

    start_inputs(0)
    start_inputs(1)
    for i in range(NCH):
        p = i % 2
        b = base0 + i * CH
        gdescs[p].wait()
        for d in wdescs[p]:
            d.wait()
        if odescs[p] is not None:
            odescs[p].wait()
        rv, xv, yv, zv, ov = rows_v[p], xw_v[p], yw_v[p], zw_v[p], out_v[p]

        def blend(o, _rv=rv, _xv=xv, _yv=yv, _zv=zv, _ov=ov):
            xw = _xv[pl.ds(o, 16)]
            yw = _yv[pl.ds(o, 16)]
            zw = _zv[pl.ds(o, 16)]
            pid = lax.iota(jnp.int32, 16) + o
            v = [plsc.load_gather(_rv, [pid, jnp.full((16,), k, jnp.int32)])
                 for k in range(8)]
            m00 = v[0] + zw * (v[1] - v[0])
            m01 = v[2] + zw * (v[3] - v[2])
            m10 = v[4] + zw * (v[5] - v[4])
            m11 = v[6] + zw * (v[7] - v[6])
            n0 = m00 + yw * (m01 - m00)
            n1 = m10 + yw * (m11 - m10)
            _ov[pl.ds(o, 16)] = n0 + xw * (n1 - n0)

        plsc.parallel_loop(0, CH, 16, unroll=4)(blend)
        odescs[p] = pltpu.async_copy(ov, out_hbm.at[pl.ds(b, CH)], semO[p])
        if i + 2 < NCH:
            start_inputs(i + 2)

    for d in odescs:
        if d is not None:
            d.wait()


@functools.cache
def _sc_blend_kernel():
    return pl.kernel(
        _sc_body,
        out_type=jax.ShapeDtypeStruct((N,), jnp.float32),
        mesh=plsc.VectorSubcoreMesh(core_axis_name="c", subcore_axis_name="s"),
        scratch_types=[
            pltpu.VMEM((CH,), jnp.int32),
            pltpu.VMEM((CH,), jnp.int32),
            pltpu.VMEM((CH, 8), jnp.float32),
            pltpu.VMEM((CH, 8), jnp.float32),
            pltpu.VMEM((CH,), jnp.float32),
            pltpu.VMEM((CH,), jnp.float32),
            pltpu.VMEM((CH,), jnp.float32),
            pltpu.VMEM((CH,), jnp.float32),
            pltpu.VMEM((CH,), jnp.float32),
            pltpu.VMEM((CH,), jnp.float32),
            pltpu.VMEM((CH,), jnp.float32),
            pltpu.VMEM((CH,), jnp.float32),
            pltpu.SemaphoreType.DMA,
            pltpu.SemaphoreType.DMA,
            pltpu.SemaphoreType.DMA,
            pltpu.SemaphoreType.DMA,
            pltpu.SemaphoreType.DMA,
            pltpu.SemaphoreType.DMA,
        ],
        compiler_params=pltpu.CompilerParams(needs_layout_passes=False,
                                             use_tc_tiling_on_sc=False),
    )


def kernel(im, defgrad, affine):
    table = _build_table(im)
    d0 = defgrad[..., 0]
    d1 = defgrad[..., 1]
    d2 = defgrad[..., 2]
    gx, gy, gz, xw, yw, zw, idx = _grid_fields(affine, d0, d1, d2)
    out = _sc_blend_kernel()(table.reshape(N, 8), idx.reshape(N),
                             xw.reshape(N), yw.reshape(N), zw.reshape(N))
    grid = jnp.stack([gx, gy, gz], axis=-1)
    return out.reshape(B, X, Y, Z, 1), grid
